# Initial kernel scaffold; baseline (speedup 1.0000x reference)
import jax
import jax.numpy as jnp
from jax.experimental import pallas as pl


def kernel(feat_s, feat_t, memory, Ws_w, Ws_b, Wt_w, Wt_b, labels, idx, contrast_idx):
    raise NotImplementedError("write your pallas kernel here")


# speedup vs baseline: 5.6406x; 5.6406x over previous
import functools

import jax
import jax.numpy as jnp
from jax import lax
from jax.experimental import pallas as pl
from jax.experimental.pallas import tpu as pltpu
from jax.experimental.pallas import tpu_sc as plsc

_BSZ = 128
_MIX = 2
_FEAT = 128
_K = 2048
_TEMP = 0.07
_EPS = 0.1
_MOM = 0.5
_ITERS = 100

_NW = 32
_TOT = _BSZ * _K
_PER_W = _TOT // _NW
_CH = 128
_NCH = _PER_W // _CH


def _sc_gather_body(mem_hbm, cidx_hbm, idx_hbm, w0_hbm, midx_hbm,
                    idxbuf, rows, sem):
    c = lax.axis_index("c")
    s = lax.axis_index("s")
    wid = s * 2 + c
    base = wid * _PER_W

    def chunk(i, carry):
        off = pl.multiple_of(base + i * _CH, _CH)
        pltpu.sync_copy(cidx_hbm.at[pl.ds(off, _CH)], idxbuf)
        pltpu.async_copy(mem_hbm.at[idxbuf], rows, sem).wait()
        pltpu.sync_copy(rows, w0_hbm.at[pl.ds(off, _CH)])
        return carry

    lax.fori_loop(0, _NCH, chunk, 0)

    @pl.when(wid == 0)
    def _():
        pltpu.sync_copy(idx_hbm, idxbuf)
        pltpu.async_copy(mem_hbm.at[idxbuf], rows, sem).wait()
        pltpu.sync_copy(rows, midx_hbm)


def _sc_gather(memory, cidx_flat, idx):
    mesh = plsc.VectorSubcoreMesh(core_axis_name="c", subcore_axis_name="s")
    f = pl.kernel(
        _sc_gather_body,
        mesh=mesh,
        out_type=[
            jax.ShapeDtypeStruct((_TOT, _FEAT), jnp.float32),
            jax.ShapeDtypeStruct((_BSZ, _FEAT), jnp.float32),
        ],
        scratch_types=[
            pltpu.VMEM((_CH,), jnp.int32),
            pltpu.VMEM((_CH, _FEAT), jnp.float32),
            pltpu.SemaphoreType.DMA,
        ],
    )
    return f(memory, cidx_flat, idx)


def _dotT(a, b):
    return lax.dot_general(a, b, (((1,), (1,)), ((), ())),
                           preferred_element_type=jnp.float32)


def _tca_body(fs_ref, ft_ref, wsw_ref, wsb_ref, wtw_ref, wtb_ref,
              midx_ref, idxr_ref, idxc_ref,
              es_ref, d_ref, posx_ref):
    fs = fs_ref[...]
    ft = ft_ref[...]
    es = _dotT(fs, wsw_ref[...]) + wsb_ref[...]
    et = _dotT(ft, wtw_ref[...]) + wtb_ref[...]
    es = es * jax.lax.rsqrt(jnp.sum(es * es, axis=1, keepdims=True))
    et = et * jax.lax.rsqrt(jnp.sum(et * et, axis=1, keepdims=True))

    ii = lax.broadcasted_iota(jnp.int32, (_BSZ, _BSZ * _MIX), 0)
    jj = lax.broadcasted_iota(jnp.int32, (_BSZ, _BSZ * _MIX), 1)
    sel_e = (jj == 2 * ii).astype(jnp.float32)
    sel_o = (jj == 2 * ii + 1).astype(jnp.float32)
    es_e = lax.dot_general(sel_e, es, (((1,), (0,)), ((), ())),
                           preferred_element_type=jnp.float32)
    es_o = lax.dot_general(sel_o, es, (((1,), (0,)), ((), ())),
                           preferred_element_type=jnp.float32)
    et_e = lax.dot_general(sel_e, et, (((1,), (0,)), ((), ())),
                           preferred_element_type=jnp.float32)
    et_o = lax.dot_general(sel_o, et, (((1,), (0,)), ((), ())),
                           preferred_element_type=jnp.float32)

    midx = midx_ref[...]
    pos = midx * _MOM + et_e * (1.0 - _MOM)
    pos = pos * jax.lax.rsqrt(jnp.sum(pos * pos, axis=1, keepdims=True))

    idx_r = idxr_ref[...]
    idx_c = idxc_ref[...]
    eqm = (idx_c == idx_r).astype(jnp.float32)
    pp = lax.broadcasted_iota(jnp.int32, (_BSZ, _BSZ), 0)
    qq = lax.broadcasted_iota(jnp.int32, (_BSZ, _BSZ), 1)
    later_dup = eqm * (qq > pp).astype(jnp.float32)
    active = 1.0 - jnp.max(later_dup, axis=1, keepdims=True)

    delta = (pos - midx) * active
    d_ref[...] = _dotT(es, delta)

    g00 = jnp.sum(es_e * et_e, axis=1, keepdims=True)
    g01 = jnp.sum(es_e * et_o, axis=1, keepdims=True)
    g10 = jnp.sum(es_o * et_e, axis=1, keepdims=True)
    g11 = jnp.sum(es_o * et_o, axis=1, keepdims=True)
    c00 = 2.0 - 2.0 * g00
    c01 = 2.0 - 2.0 * g01
    c10 = 2.0 - 2.0 * g10
    c11 = 2.0 - 2.0 * g11
    lmu = jnp.log(0.5 + 1e-8)

    def m_all(u0, u1, v0, v1):
        m00 = (-c00 + u0 + v0) / _EPS
        m01 = (-c01 + u0 + v1) / _EPS
        m10 = (-c10 + u1 + v0) / _EPS
        m11 = (-c11 + u1 + v1) / _EPS
        return m00, m01, m10, m11

    def sink_step(_, carry):
        u0, u1, v0, v1 = carry
        m00, m01, m10, m11 = m_all(u0, u1, v0, v1)
        u0 = _EPS * (lmu - jnp.logaddexp(m00, m01)) + u0
        u1 = _EPS * (lmu - jnp.logaddexp(m10, m11)) + u1
        m00, m01, m10, m11 = m_all(u0, u1, v0, v1)
        v0 = _EPS * (lmu - jnp.logaddexp(m00, m10)) + v0
        v1 = _EPS * (lmu - jnp.logaddexp(m01, m11)) + v1
        return u0, u1, v0, v1

    z = jnp.zeros((_BSZ, 1), jnp.float32)
    u0, u1, v0, v1 = lax.fori_loop(0, _ITERS, sink_step, (z, z, z, z))
    m00, m01, m10, m11 = m_all(u0, u1, v0, v1)
    posx = (jnp.exp(m00) * g00 + jnp.exp(m01) * g01 +
            jnp.exp(m10) * g10 + jnp.exp(m11) * g11)

    es_ref[...] = es
    posx_ref[...] = posx


def _tca(feat_s, feat_t, wsw, wsb, wtw, wtb, midx, idx_row, idx_col):
    return pl.pallas_call(
        _tca_body,
        out_shape=[
            jax.ShapeDtypeStruct((_BSZ * _MIX, _FEAT), jnp.float32),
            jax.ShapeDtypeStruct((_BSZ * _MIX, _BSZ), jnp.float32),
            jax.ShapeDtypeStruct((_BSZ, 1), jnp.float32),
        ],
    )(feat_s, feat_t, wsw, wsb, wtw, wtb, midx, idx_row, idx_col)


def _tcb_body(w0_ref, es2_ref, d2_ref, cidx_ref, idxc_ref, ng_ref):
    b = pl.program_id(0)
    w0 = w0_ref[0]
    es_pair = es2_ref[0]
    d_pair = d2_ref[0]
    cidx_row = cidx_ref[0]
    idx_col = idxc_ref[...]

    neg = _dotT(es_pair, w0)
    eq = (idx_col == cidx_row).astype(jnp.float32)
    corr = lax.dot_general(d_pair, eq, (((1,), (0,)), ((), ())),
                           preferred_element_type=jnp.float32)
    ng = jnp.exp((neg + corr) / _TEMP)
    ng_ref[0, b] = jnp.sum(ng)


def _tcb(w0r, es2, d2, cidx3, idx_col):
    return pl.pallas_call(
        _tcb_body,
        grid=(_BSZ,),
        in_specs=[
            pl.BlockSpec((1, _K, _FEAT), lambda b: (b, 0, 0)),
            pl.BlockSpec((1, _MIX, _FEAT), lambda b: (b, 0, 0)),
            pl.BlockSpec((1, _MIX, _BSZ), lambda b: (b, 0, 0)),
            pl.BlockSpec((1, 1, _K), lambda b: (b, 0, 0)),
            pl.BlockSpec((_BSZ, 1), lambda b: (0, 0)),
        ],
        out_specs=pl.BlockSpec(memory_space=pltpu.SMEM),
        out_shape=jax.ShapeDtypeStruct((1, _BSZ), jnp.float32),
    )(w0r, es2, d2, cidx3, idx_col)


def _tcc_body(posx_ref, ng_ref, out_ref):
    p = jnp.exp(posx_ref[...] / _TEMP)
    ngs = ng_ref[...]
    logits = jnp.log(p / (p + ngs))
    out_ref[0, 0] = -jnp.sum(logits) / _BSZ


def _tcc(posx_row, ng_row):
    return pl.pallas_call(
        _tcc_body,
        out_specs=pl.BlockSpec(memory_space=pltpu.SMEM),
        out_shape=jax.ShapeDtypeStruct((1, 1), jnp.float32),
    )(posx_row, ng_row)


def kernel(feat_s, feat_t, memory, Ws_w, Ws_b, Wt_w, Wt_b, labels, idx,
           contrast_idx):
    feat_s = feat_s.reshape(_BSZ * _MIX, -1)
    feat_t = feat_t.reshape(_BSZ * _MIX, -1)
    cidx_flat = contrast_idx.reshape(_TOT).astype(jnp.int32)
    idx_i = idx.astype(jnp.int32)

    w0, midx = _sc_gather(memory, cidx_flat, idx_i)

    idx_f = idx.astype(jnp.float32)
    idx_row = idx_f.reshape(1, _BSZ)
    idx_col = idx_f.reshape(_BSZ, 1)
    es, dmat, posx = _tca(feat_s, feat_t, Ws_w, Ws_b.reshape(1, _FEAT),
                          Wt_w, Wt_b.reshape(1, _FEAT), midx, idx_row, idx_col)

    w0r = w0.reshape(_BSZ, _K, _FEAT)
    es2 = es.reshape(_BSZ, _MIX, _FEAT)
    d2 = dmat.reshape(_BSZ, _MIX, _BSZ)
    cidx3 = contrast_idx.astype(jnp.float32).reshape(_BSZ, 1, _K)
    ng_row = _tcb(w0r, es2, d2, cidx3, idx_col)

    loss = _tcc(posx.reshape(1, _BSZ), ng_row)
    return loss.reshape(())
